# Initial kernel scaffold; baseline (speedup 1.0000x reference)
import jax
import jax.numpy as jnp
from jax.experimental import pallas as pl


def kernel(x, lut):
    raise NotImplementedError("write your pallas kernel here")


# speedup vs baseline: 1.3425x; 1.3425x over previous
import functools
import math

import jax
import jax.numpy as jnp
from jax import lax
from jax.experimental import pallas as pl
from jax.experimental.pallas import tpu as pltpu
from jax.experimental.pallas import tpu_sc as plsc

D_INPUT = 100000
D_MODEL = 1024
BATCH = 4
SEQ = 4096
SCALE = math.sqrt(D_MODEL)

NC = 2
NS = 16
NW = NC * NS
L = 16

B_TOTAL = BATCH * SEQ
BPW = B_TOTAL // NW
CHUNK = 32
NCHUNK = BPW // CHUNK
VPR = D_MODEL // L

_mesh = plsc.VectorSubcoreMesh(core_axis_name="c", subcore_axis_name="s")


@functools.partial(
    pl.kernel,
    out_type=jax.ShapeDtypeStruct((B_TOTAL, D_MODEL), jnp.float32),
    mesh=_mesh,
    scratch_types=[
        pltpu.VMEM((BPW,), jnp.int32),
        pltpu.VMEM((CHUNK, D_MODEL), jnp.float32),
        pltpu.VMEM((CHUNK, D_MODEL), jnp.float32),
        pltpu.SemaphoreType.DMA,
        pltpu.SemaphoreType.DMA,
        pltpu.SemaphoreType.DMA,
        pltpu.SemaphoreType.DMA,
    ],
)
def _emb_lookup(x_hbm, lut_hbm, out_hbm, idx_v, buf0, buf1,
                gsem0, gsem1, ssem0, ssem1):
    wid = lax.axis_index("s") * NC + lax.axis_index("c")
    base = wid * BPW
    bufs = (buf0, buf1)
    gsems = (gsem0, gsem1)
    ssems = (ssem0, ssem1)

    pltpu.sync_copy(x_hbm.at[pl.ds(base, BPW)], idx_v)

    def start_gather(j):
        return pltpu.async_copy(
            lut_hbm.at[idx_v.at[pl.ds(j * CHUNK, CHUNK)]],
            bufs[j % 2], gsems[j % 2])

    def scale_chunk(buf):
        def row_body(r, _):
            for c in range(VPR):
                sl = pl.ds(c * L, L)
                buf[r, sl] = buf[r, sl] * SCALE
            return 0
        lax.fori_loop(0, CHUNK, row_body, 0, unroll=False)

    gathers = [None] * NCHUNK
    stores = [None] * NCHUNK
    gathers[0] = start_gather(0)
    for j in range(NCHUNK):
        p = j % 2
        gathers[j].wait()
        if j + 1 < NCHUNK:
            if j >= 1:
                stores[j - 1].wait()
            gathers[j + 1] = start_gather(j + 1)
        scale_chunk(bufs[p])
        stores[j] = pltpu.async_copy(
            bufs[p], out_hbm.at[pl.ds(base + j * CHUNK, CHUNK)], ssems[p])
    stores[NCHUNK - 2].wait()
    stores[NCHUNK - 1].wait()


def kernel(x, lut):
    out = _emb_lookup(x.reshape(B_TOTAL).astype(jnp.int32), lut)
    return out.reshape(BATCH, SEQ, D_MODEL)


# speedup vs baseline: 1.4181x; 1.0563x over previous
import functools
import math

import jax
import jax.numpy as jnp
from jax import lax
from jax.experimental import pallas as pl
from jax.experimental.pallas import tpu as pltpu
from jax.experimental.pallas import tpu_sc as plsc

D_INPUT = 100000
D_MODEL = 1024
BATCH = 4
SEQ = 4096
SCALE = math.sqrt(D_MODEL)

NC = 2
NS = 16
NW = NC * NS
L = 16

B_TOTAL = BATCH * SEQ
BPW = B_TOTAL // NW
CHUNK = 32
NCHUNK = BPW // CHUNK
VPR = D_MODEL // L

_mesh = plsc.VectorSubcoreMesh(core_axis_name="c", subcore_axis_name="s")


@functools.partial(
    pl.kernel,
    out_type=jax.ShapeDtypeStruct((B_TOTAL, D_MODEL), jnp.float32),
    mesh=_mesh,
    scratch_types=[
        pltpu.VMEM((BPW,), jnp.int32),
        pltpu.VMEM((CHUNK, D_MODEL), jnp.float32),
        pltpu.VMEM((CHUNK, D_MODEL), jnp.float32),
        pltpu.VMEM((CHUNK, D_MODEL), jnp.float32),
        pltpu.SemaphoreType.DMA,
        pltpu.SemaphoreType.DMA,
        pltpu.SemaphoreType.DMA,
        pltpu.SemaphoreType.DMA,
        pltpu.SemaphoreType.DMA,
        pltpu.SemaphoreType.DMA,
    ],
)
def _emb_lookup(x_hbm, lut_hbm, out_hbm, idx_v, buf0, buf1, buf2,
                gsem0, gsem1, gsem2, ssem0, ssem1, ssem2):
    wid = lax.axis_index("s") * NC + lax.axis_index("c")
    base = wid * BPW
    bufs = (buf0, buf1, buf2)
    gsems = (gsem0, gsem1, gsem2)
    ssems = (ssem0, ssem1, ssem2)

    pltpu.sync_copy(x_hbm.at[pl.ds(base, BPW)], idx_v)

    def start_gather(j):
        return pltpu.async_copy(
            lut_hbm.at[idx_v.at[pl.ds(j * CHUNK, CHUNK)]],
            bufs[j % 3], gsems[j % 3])

    def scale_chunk(buf):
        def row_body(r, _):
            for c in range(VPR):
                sl = pl.ds(c * L, L)
                buf[r, sl] = buf[r, sl] * SCALE
            return 0
        lax.fori_loop(0, CHUNK, row_body, 0, unroll=False)

    gathers = [None] * NCHUNK
    stores = [None] * NCHUNK
    gathers[0] = start_gather(0)
    gathers[1] = start_gather(1)
    for j in range(NCHUNK):
        p = j % 3
        gathers[j].wait()
        if j + 2 < NCHUNK:
            if j >= 1:
                stores[j - 1].wait()
            gathers[j + 2] = start_gather(j + 2)
        scale_chunk(bufs[p])
        stores[j] = pltpu.async_copy(
            bufs[p], out_hbm.at[pl.ds(base + j * CHUNK, CHUNK)], ssems[p])
    stores[NCHUNK - 2].wait()
    stores[NCHUNK - 1].wait()


def kernel(x, lut):
    out = _emb_lookup(x.reshape(B_TOTAL).astype(jnp.int32), lut)
    return out.reshape(BATCH, SEQ, D_MODEL)


# speedup vs baseline: 1.4610x; 1.0303x over previous
import functools
import math

import jax
import jax.numpy as jnp
from jax import lax
from jax.experimental import pallas as pl
from jax.experimental.pallas import tpu as pltpu
from jax.experimental.pallas import tpu_sc as plsc

D_INPUT = 100000
D_MODEL = 1024
BATCH = 4
SEQ = 4096
SCALE = math.sqrt(D_MODEL)

NC = 2
NS = 16
NW = NC * NS
L = 16

B_TOTAL = BATCH * SEQ
BPW = B_TOTAL // NW
CHUNK = 16
NCHUNK = BPW // CHUNK
VPR = D_MODEL // L
NBUF = 7
AHEAD = 5

_mesh = plsc.VectorSubcoreMesh(core_axis_name="c", subcore_axis_name="s")


@functools.partial(
    pl.kernel,
    out_type=jax.ShapeDtypeStruct((B_TOTAL, D_MODEL), jnp.float32),
    mesh=_mesh,
    scratch_types=(
        [pltpu.VMEM((BPW,), jnp.int32)]
        + [pltpu.VMEM((CHUNK, D_MODEL), jnp.float32)] * NBUF
        + [pltpu.SemaphoreType.DMA] * (2 * NBUF)
    ),
)
def _emb_lookup(x_hbm, lut_hbm, out_hbm, idx_v, *rest):
    bufs = rest[:NBUF]
    gsems = rest[NBUF:2 * NBUF]
    ssems = rest[2 * NBUF:]
    wid = lax.axis_index("s") * NC + lax.axis_index("c")
    base = wid * BPW

    pltpu.sync_copy(x_hbm.at[pl.ds(base, BPW)], idx_v)

    def start_gather(j):
        return pltpu.async_copy(
            lut_hbm.at[idx_v.at[pl.ds(j * CHUNK, CHUNK)]],
            bufs[j % NBUF], gsems[j % NBUF])

    def scale_chunk(buf):
        def row_body(r, _):
            for c in range(VPR):
                sl = pl.ds(c * L, L)
                buf[r, sl] = buf[r, sl] * SCALE
            return 0
        lax.fori_loop(0, CHUNK, row_body, 0, unroll=False)

    gathers = [None] * NCHUNK
    stores = [None] * NCHUNK
    for j in range(AHEAD):
        gathers[j] = start_gather(j)
    for j in range(NCHUNK):
        p = j % NBUF
        if j + AHEAD < NCHUNK:
            prev = j + AHEAD - NBUF
            if prev >= 0:
                stores[prev].wait()
            gathers[j + AHEAD] = start_gather(j + AHEAD)
        gathers[j].wait()
        scale_chunk(bufs[p])
        stores[j] = pltpu.async_copy(
            bufs[p], out_hbm.at[pl.ds(base + j * CHUNK, CHUNK)], ssems[p])
    for j in range(max(0, NCHUNK - NBUF), NCHUNK):
        stores[j].wait()


def kernel(x, lut):
    out = _emb_lookup(x.reshape(B_TOTAL).astype(jnp.int32), lut)
    return out.reshape(BATCH, SEQ, D_MODEL)


# speedup vs baseline: 1.6685x; 1.1420x over previous
import functools
import math

import jax
import jax.numpy as jnp
from jax import lax
from jax.experimental import pallas as pl
from jax.experimental.pallas import tpu as pltpu
from jax.experimental.pallas import tpu_sc as plsc

D_INPUT = 100000
D_MODEL = 1024
BATCH = 4
SEQ = 4096
SCALE = math.sqrt(D_MODEL)

NC = 2
NS = 16
NW = NC * NS
L = 16

B_TOTAL = BATCH * SEQ
BPW = B_TOTAL // NW
CHUNK = 8
NCHUNK = BPW // CHUNK
VPR = D_MODEL // L
NBUF = 8
AHEAD = 6

_mesh = plsc.VectorSubcoreMesh(core_axis_name="c", subcore_axis_name="s")


@functools.partial(
    pl.kernel,
    out_type=jax.ShapeDtypeStruct((B_TOTAL, D_MODEL), jnp.float32),
    mesh=_mesh,
    scratch_types=[
        pltpu.VMEM((BPW,), jnp.int32),
        pltpu.VMEM((NBUF * CHUNK, D_MODEL), jnp.float32),
        pltpu.SemaphoreType.DMA((NBUF,)),
        pltpu.SemaphoreType.DMA((NBUF,)),
    ],
)
def _emb_lookup(x_hbm, lut_hbm, out_hbm, idx_v, ring, gsem, ssem):
    wid = lax.axis_index("s") * NC + lax.axis_index("c")
    base = wid * BPW

    pltpu.sync_copy(x_hbm.at[pl.ds(base, BPW)], idx_v)

    def gather_descr(j, p):
        return pltpu.make_async_copy(
            lut_hbm.at[idx_v.at[pl.ds(j * CHUNK, CHUNK)]],
            ring.at[pl.ds(p * CHUNK, CHUNK)],
            gsem.at[p])

    def store_descr(j, p):
        return pltpu.make_async_copy(
            ring.at[pl.ds(p * CHUNK, CHUNK)],
            out_hbm.at[pl.ds(base + j * CHUNK, CHUNK)],
            ssem.at[p])

    for j in range(AHEAD):
        gather_descr(j, j % NBUF).start()

    def step(j, _):
        p = lax.rem(j, NBUF)
        ja = j + AHEAD
        q = lax.rem(ja, NBUF)

        @pl.when(jnp.logical_and(ja < NCHUNK, ja >= NBUF))
        def _():
            store_descr(ja - NBUF, q).wait()

        @pl.when(ja < NCHUNK)
        def _():
            gather_descr(ja, q).start()

        gather_descr(j, p).wait()
        for r in range(CHUNK):
            for c in range(VPR):
                sl = pl.ds(c * L, L)
                ring[p * CHUNK + r, sl] = ring[p * CHUNK + r, sl] * SCALE
        store_descr(j, p).start()
        return 0

    lax.fori_loop(0, NCHUNK, step, 0, unroll=False)

    for p in range(NBUF):
        store_descr(NCHUNK - NBUF + p, p).wait()


def kernel(x, lut):
    out = _emb_lookup(x.reshape(B_TOTAL).astype(jnp.int32), lut)
    return out.reshape(BATCH, SEQ, D_MODEL)
